# Initial kernel scaffold; baseline (speedup 1.0000x reference)
import jax
import jax.numpy as jnp
from jax.experimental import pallas as pl


def kernel(x, node_type, sensor_id, edge_index, edge_attr, type_emb_t, sensor_emb_t, enc_W1, enc_b1, enc_W2, enc_b2, g1_Wl, g1_bl, g1_Wr, g1_br, g1_We, g1_att, g1_bias, g2_Wl, g2_bl, g2_Wr, g2_br, g2_We, g2_att, g2_bias, gru_Wi, gru_bi, gru_Wh, gru_bh, dec_W1, dec_b1, dec_W2, dec_b2):
    raise NotImplementedError("write your pallas kernel here")


# speedup vs baseline: 1.0268x; 1.0268x over previous
import functools

import jax
import jax.numpy as jnp
from jax import lax
from jax.experimental import pallas as pl
from jax.experimental.pallas import tpu as pltpu

_N = 50000
_E = 800000
_HID = 64
_HEADS = 4
_C = 16
_NB = 2000
_EB = 8000


def _enc_proj_body(x_ref, nt_ref, sid_ref, te_ref, se_ref, w1_ref, b1_ref,
                   w2_ref, b2_ref, wl_ref, bl_ref, wr_ref, br_ref,
                   xl_ref, xr_ref):
    x = x_ref[...]
    nt = nt_ref[...]
    sid = sid_ref[...]
    w1 = w1_ref[...]
    temb = te_ref[...]
    semb = se_ref[...]
    t_rows = jnp.dot(temb, w1[7:15, :], preferred_element_type=jnp.float32)
    s_rows = jnp.dot(semb, w1[15:23, :], preferred_element_type=jnp.float32)
    te = jnp.where(nt == 0, t_rows[0][None, :], t_rows[1][None, :])
    se = jnp.where(
        sid <= 1,
        jnp.where(sid == 0, s_rows[0][None, :], s_rows[1][None, :]),
        jnp.where(sid == 2, s_rows[2][None, :], s_rows[3][None, :]))
    h1 = jnp.dot(x, w1[0:7, :], preferred_element_type=jnp.float32)
    h1 = jax.nn.relu(h1 + te + se + b1_ref[...])
    h0 = jnp.dot(h1, w2_ref[...], preferred_element_type=jnp.float32) + b2_ref[...]
    xl = jnp.dot(h0, wl_ref[...], preferred_element_type=jnp.float32) + bl_ref[...]
    xr = jnp.dot(h0, wr_ref[...], preferred_element_type=jnp.float32) + br_ref[...]
    xl_ref[0, :, :] = xl[:, 0:32]
    xl_ref[1, :, :] = xl[:, 32:64]
    xr_ref[0, :, :] = xr[:, 0:32]
    xr_ref[1, :, :] = xr[:, 32:64]


def _full(shape):
    return pl.BlockSpec(shape, lambda i: tuple(0 for _ in shape))


def _encode_proj(x, node_type, sensor_id, type_emb_t, sensor_emb_t,
                 enc_W1, enc_b1, enc_W2, enc_b2, Wl, bl, Wr, br):
    grid = _N // _NB
    hp = jax.ShapeDtypeStruct((2, _N, 32), jnp.float32)
    return pl.pallas_call(
        _enc_proj_body,
        grid=(grid,),
        in_specs=[
            pl.BlockSpec((_NB, 7), lambda i: (i, 0)),
            pl.BlockSpec((_NB, 1), lambda i: (i, 0)),
            pl.BlockSpec((_NB, 1), lambda i: (i, 0)),
            _full((2, 8)), _full((4, 8)),
            _full((23, _HID)), _full((1, _HID)),
            _full((_HID, _HID)), _full((1, _HID)),
            _full((_HID, _HID)), _full((1, _HID)),
            _full((_HID, _HID)), _full((1, _HID)),
        ],
        out_specs=[
            pl.BlockSpec((2, _NB, 32), lambda i: (0, i, 0)),
            pl.BlockSpec((2, _NB, 32), lambda i: (0, i, 0)),
        ],
        out_shape=[hp, hp],
    )(x, node_type.reshape(_N, 1).astype(jnp.int32),
      sensor_id.reshape(_N, 1).astype(jnp.int32),
      type_emb_t, sensor_emb_t,
      enc_W1, enc_b1.reshape(1, _HID), enc_W2, enc_b2.reshape(1, _HID),
      Wl, bl.reshape(1, _HID), Wr, br.reshape(1, _HID))


def _ea_body(eattr_ref, we1_ref, we2_ref, ea1_ref, ea2_ref):
    ea = eattr_ref[...]
    e1 = jnp.dot(ea, we1_ref[...], preferred_element_type=jnp.float32)
    e2 = jnp.dot(ea, we2_ref[...], preferred_element_type=jnp.float32)
    ea1_ref[0, :, :] = e1[:, 0:32]
    ea1_ref[1, :, :] = e1[:, 32:64]
    ea2_ref[0, :, :] = e2[:, 0:32]
    ea2_ref[1, :, :] = e2[:, 32:64]


def _edge_proj(edge_attr, We1, We2):
    grid = _E // _EB
    hp = jax.ShapeDtypeStruct((2, _E, 32), jnp.float32)
    return pl.pallas_call(
        _ea_body,
        grid=(grid,),
        in_specs=[
            pl.BlockSpec((_EB, 6), lambda i: (i, 0)),
            _full((6, _HID)), _full((6, _HID)),
        ],
        out_specs=[
            pl.BlockSpec((2, _EB, 32), lambda i: (0, i, 0)),
            pl.BlockSpec((2, _EB, 32), lambda i: (0, i, 0)),
        ],
        out_shape=[hp, hp],
    )(edge_attr, We1, We2)


def _mid_body(o_ref, bias_ref, wl_ref, bl_ref, wr_ref, br_ref,
              xl_ref, xr_ref):
    h = jnp.concatenate([o_ref[0, :, :], o_ref[1, :, :]], axis=1)
    h = jax.nn.relu(h + bias_ref[...])
    xl = jnp.dot(h, wl_ref[...], preferred_element_type=jnp.float32) + bl_ref[...]
    xr = jnp.dot(h, wr_ref[...], preferred_element_type=jnp.float32) + br_ref[...]
    xl_ref[0, :, :] = xl[:, 0:32]
    xl_ref[1, :, :] = xl[:, 32:64]
    xr_ref[0, :, :] = xr[:, 0:32]
    xr_ref[1, :, :] = xr[:, 32:64]


def _mid_proj(o1, g1_bias, Wl, bl, Wr, br):
    grid = _N // _NB
    hp = jax.ShapeDtypeStruct((2, _N, 32), jnp.float32)
    return pl.pallas_call(
        _mid_body,
        grid=(grid,),
        in_specs=[
            pl.BlockSpec((2, _NB, 32), lambda i: (0, i, 0)),
            _full((1, _HID)),
            _full((_HID, _HID)), _full((1, _HID)),
            _full((_HID, _HID)), _full((1, _HID)),
        ],
        out_specs=[
            pl.BlockSpec((2, _NB, 32), lambda i: (0, i, 0)),
            pl.BlockSpec((2, _NB, 32), lambda i: (0, i, 0)),
        ],
        out_shape=[hp, hp],
    )(o1, g1_bias.reshape(1, _HID), Wl, bl.reshape(1, _HID),
      Wr, br.reshape(1, _HID))


def _final_body(o_ref, bias_ref, wi_ref, bi_ref, bh_ref,
                dw1_ref, db1_ref, dw2_ref, db2_ref, out_ref, nh_ref):
    h = jnp.concatenate([o_ref[0, :, :], o_ref[1, :, :]], axis=1)
    h = h + bias_ref[...]
    gi = jnp.dot(h, wi_ref[...], preferred_element_type=jnp.float32) + bi_ref[...]
    bh = bh_ref[...]
    r = jax.nn.sigmoid(gi[:, 0:64] + bh[:, 0:64])
    z = jax.nn.sigmoid(gi[:, 64:128] + bh[:, 64:128])
    n = jnp.tanh(gi[:, 128:192] + r * bh[:, 128:192])
    nh = (1.0 - z) * n
    d1 = jax.nn.relu(
        jnp.dot(nh, dw1_ref[...], preferred_element_type=jnp.float32)
        + db1_ref[...])
    out = jnp.dot(d1, dw2_ref[...], preferred_element_type=jnp.float32) + db2_ref[...]
    out_ref[...] = out
    nh_ref[...] = nh


def _final(o2, g2_bias, gru_Wi, gru_bi, gru_bh, dec_W1, dec_b1,
           dec_W2, dec_b2):
    grid = _N // _NB
    return pl.pallas_call(
        _final_body,
        grid=(grid,),
        in_specs=[
            pl.BlockSpec((2, _NB, 32), lambda i: (0, i, 0)),
            _full((1, _HID)),
            _full((_HID, 3 * _HID)), _full((1, 3 * _HID)),
            _full((1, 3 * _HID)),
            _full((_HID, _HID)), _full((1, _HID)),
            _full((_HID, 7)), _full((1, 7)),
        ],
        out_specs=[
            pl.BlockSpec((_NB, 7), lambda i: (i, 0)),
            pl.BlockSpec((_NB, _HID), lambda i: (i, 0)),
        ],
        out_shape=[
            jax.ShapeDtypeStruct((_N, 7), jnp.float32),
            jax.ShapeDtypeStruct((_N, _HID), jnp.float32),
        ],
    )(o2, g2_bias.reshape(1, _HID), gru_Wi, gru_bi.reshape(1, 3 * _HID),
      gru_bh.reshape(1, 3 * _HID), dec_W1, dec_b1.reshape(1, _HID),
      dec_W2, dec_b2.reshape(1, 7))


def _edge_phase_xla(xl_hp, xr_hp, ea_hp, src, dst, att):
    xl = jnp.concatenate([xl_hp[0], xl_hp[1]], axis=1).reshape(_N, _HEADS, _C)
    xr = jnp.concatenate([xr_hp[0], xr_hp[1]], axis=1).reshape(_N, _HEADS, _C)
    ea = jnp.concatenate([ea_hp[0], ea_hp[1]], axis=1).reshape(_E, _HEADS, _C)
    z = xl[src] + xr[dst] + ea
    z = jnp.maximum(z, 0.2 * z)
    logit = jnp.sum(z * att[None, :, :], axis=-1)
    e = jnp.exp(logit)
    s = jax.ops.segment_sum(e, dst, num_segments=_N)
    alpha = e / (s[dst] + 1e-16)
    msg = xl[src] * alpha[:, :, None]
    out = jax.ops.segment_sum(msg, dst, num_segments=_N)
    out = out.reshape(_N, _HID)
    o_hp = jnp.stack([out[:, 0:32], out[:, 32:64]], axis=0)
    return o_hp, alpha


def kernel(x, node_type, sensor_id, edge_index, edge_attr, type_emb_t,
           sensor_emb_t, enc_W1, enc_b1, enc_W2, enc_b2, g1_Wl, g1_bl,
           g1_Wr, g1_br, g1_We, g1_att, g1_bias, g2_Wl, g2_bl, g2_Wr,
           g2_br, g2_We, g2_att, g2_bias, gru_Wi, gru_bi, gru_Wh, gru_bh,
           dec_W1, dec_b1, dec_W2, dec_b2):
    src = edge_index[0].astype(jnp.int32)
    dst = edge_index[1].astype(jnp.int32)

    xl1, xr1 = _encode_proj(x, node_type, sensor_id, type_emb_t,
                            sensor_emb_t, enc_W1, enc_b1, enc_W2, enc_b2,
                            g1_Wl, g1_bl, g1_Wr, g1_br)
    ea1, ea2 = _edge_proj(edge_attr, g1_We, g2_We)

    o1, a1 = _edge_phase_xla(xl1, xr1, ea1, src, dst, g1_att)

    xl2, xr2 = _mid_proj(o1, g1_bias, g2_Wl, g2_bl, g2_Wr, g2_br)

    o2, a2 = _edge_phase_xla(xl2, xr2, ea2, src, dst, g2_att)

    out, nh = _final(o2, g2_bias, gru_Wi, gru_bi, gru_bh, dec_W1, dec_b1,
                     dec_W2, dec_b2)
    return out, nh, a1, a2
